# baseline (speedup 1.0000x reference)
import functools

import jax
import jax.numpy as jnp
from jax import lax
from jax.experimental import pallas as pl
from jax.experimental.pallas import tpu as pltpu
from jax.experimental.pallas import tpu_sc as plsc

N = 65536
D = 1024
K = 16384
TPR = 256
TOTAL = K + N // TPR
MIN32 = -(2**31)

NC = 2
NS = 16
NW = NC * NS
RPW = TOTAL // NW
CHUNK = 40
NCHUNK = RPW // CHUNK


def _select_kernel(scores_ref, pos_ref, val_ref):
    s = scores_ref[...]
    b = lax.bitcast_convert_type(s, jnp.int32)
    skey = jnp.where(b < 0, jnp.bitwise_not(b) ^ MIN32, b)

    def body(i, p):
        q = p | jnp.left_shift(jnp.int32(1), 31 - i)
        cnt = jnp.sum((skey >= (q ^ MIN32)).astype(jnp.int32))
        return jnp.where(cnt >= K, q, p)

    p = lax.fori_loop(0, 32, body, jnp.int32(0))
    ts = p ^ MIN32
    cgt = jnp.sum((skey > ts).astype(jnp.int32))
    needed = K - cgt

    eq = skey == ts
    row = lax.broadcasted_iota(jnp.int32, (512, 128), 0)
    col = lax.broadcasted_iota(jnp.int32, (512, 128), 1)
    idx = row * 128 + col

    def body2(i, m):
        cand = m + jnp.left_shift(jnp.int32(1), 15 - i)
        cnt = jnp.sum(((idx < cand) & eq).astype(jnp.int32))
        return jnp.where(cnt < needed, cand, m)

    m = lax.fori_loop(0, 16, body2, jnp.int32(0))

    mask = (skey > ts) | (eq & (idx <= m)) | (idx % TPR == TPR - 1)
    mf = mask.astype(jnp.float32)

    ku = lax.broadcasted_iota(jnp.int32, (128, 128), 0)
    lu = lax.broadcasted_iota(jnp.int32, (128, 128), 1)
    u = (ku <= lu).astype(jnp.float32)
    prow = jnp.dot(mf, u, preferred_element_type=jnp.float32)
    rs = prow[:, 127:128]
    rr = lax.broadcasted_iota(jnp.int32, (512, 512), 0)
    qq = lax.broadcasted_iota(jnp.int32, (512, 512), 1)
    lt = (qq < rr).astype(jnp.float32)
    offs = jnp.dot(lt, rs, preferred_element_type=jnp.float32)
    rank_excl = (prow + offs - mf).astype(jnp.int32)

    s_total = jnp.sum(mf).astype(jnp.int32)
    u_rank = idx - rank_excl
    pad_pos = s_total + u_rank
    pos_ref[...] = jnp.where(
        mask, rank_excl, jnp.where(pad_pos < TOTAL, pad_pos, TOTAL + (idx >> 12))
    )
    val_ref[...] = jnp.where(mask, idx, N - 1)


_select = pl.pallas_call(
    _select_kernel,
    out_shape=(
        jax.ShapeDtypeStruct((512, 128), jnp.int32),
        jax.ShapeDtypeStruct((512, 128), jnp.int32),
    ),
)


@functools.partial(
    pl.kernel,
    out_type=jax.ShapeDtypeStruct((TOTAL, D), jnp.float32),
    mesh=plsc.VectorSubcoreMesh(core_axis_name="c", subcore_axis_name="s"),
    scratch_types=[
        pltpu.VMEM((32, 128), jnp.int32),
        pltpu.VMEM((32, 128), jnp.int32),
        pltpu.VMEM((CHUNK,), jnp.int32),
        pltpu.VMEM((CHUNK,), jnp.int32),
        pltpu.VMEM((CHUNK, D), jnp.float32),
        pltpu.VMEM((CHUNK, D), jnp.float32),
        pltpu.VMEM_SHARED((TOTAL + NS,), jnp.int32),
        pltpu.SemaphoreType.DMA,
        pltpu.SemaphoreType.DMA,
        pltpu.SemaphoreType.DMA,
    ],
)
def _prune_gather(pos_hbm, val_hbm, table_hbm, out_hbm,
                  pos2d, val2d, idxc0, idxc1, rows0, rows1, idx_sh,
                  gsem, wsem0, wsem1):
    cid = lax.axis_index("c")
    sid = lax.axis_index("s")

    pltpu.sync_copy(pos_hbm.at[sid], pos2d)
    pltpu.sync_copy(val_hbm.at[sid], val2d)
    descs = [
        pltpu.async_copy(val2d.at[r], idx_sh.at[pos2d.at[r]], gsem)
        for r in range(32)
    ]
    for d in descs:
        d.wait()
    plsc.subcore_barrier()

    wid = sid * NC + cid
    base = wid * RPW
    bufs = ((idxc0, rows0, wsem0), (idxc1, rows1, wsem1))
    wb = [None, None]
    for ch in range(NCHUNK):
        b = ch & 1
        idxc, rows, wsem = bufs[b]
        if wb[b] is not None:
            wb[b].wait()
        off = base + ch * CHUNK
        pltpu.sync_copy(idx_sh.at[pl.ds(off, CHUNK)], idxc)
        pltpu.async_copy(table_hbm.at[idxc], rows, gsem).wait()
        wb[b] = pltpu.async_copy(rows, out_hbm.at[pl.ds(off, CHUNK)], wsem)
    for d in wb:
        if d is not None:
            d.wait()


def kernel(attention_scores, local_img_fea):
    pos, val = _select(attention_scores.reshape(512, 128))
    pos3 = pos.reshape(NS, 32, 128)
    val3 = val.reshape(NS, 32, 128)
    return _prune_gather(pos3, val3, local_img_fea)


# speedup vs baseline: 1.0098x; 1.0098x over previous
import functools

import jax
import jax.numpy as jnp
from jax import lax
from jax.experimental import pallas as pl
from jax.experimental.pallas import tpu as pltpu
from jax.experimental.pallas import tpu_sc as plsc

N = 65536
D = 1024
K = 16384
TPR = 256
TOTAL = K + N // TPR
MIN32 = -(2**31)

NC = 2
NS = 16
NW = NC * NS
RPW = TOTAL // NW
CHUNK = 40
NCHUNK = RPW // CHUNK


def _select_kernel(scores_ref, pos_ref, val_ref):
    s = scores_ref[...]
    b = lax.bitcast_convert_type(s, jnp.int32)
    skey = jnp.where(b < 0, jnp.bitwise_not(b) ^ MIN32, b)

    def body(i, p):
        q = p | jnp.left_shift(jnp.int32(1), 31 - i)
        cnt = jnp.sum((skey >= (q ^ MIN32)).astype(jnp.int32))
        return jnp.where(cnt >= K, q, p)

    p = lax.fori_loop(0, 32, body, jnp.int32(0))
    ts = p ^ MIN32
    cgt = jnp.sum((skey > ts).astype(jnp.int32))
    needed = K - cgt

    eq = skey == ts
    row = lax.broadcasted_iota(jnp.int32, (512, 128), 0)
    col = lax.broadcasted_iota(jnp.int32, (512, 128), 1)
    idx = row * 128 + col

    def body2(i, m):
        cand = m + jnp.left_shift(jnp.int32(1), 15 - i)
        cnt = jnp.sum(((idx < cand) & eq).astype(jnp.int32))
        return jnp.where(cnt < needed, cand, m)

    m = lax.fori_loop(0, 16, body2, jnp.int32(0))

    mask = (skey > ts) | (eq & (idx <= m)) | (idx % TPR == TPR - 1)
    mf = mask.astype(jnp.float32)

    ku = lax.broadcasted_iota(jnp.int32, (128, 128), 0)
    lu = lax.broadcasted_iota(jnp.int32, (128, 128), 1)
    u = (ku <= lu).astype(jnp.float32)
    prow = jnp.dot(mf, u, preferred_element_type=jnp.float32)
    rs = prow[:, 127:128]
    rr = lax.broadcasted_iota(jnp.int32, (512, 512), 0)
    qq = lax.broadcasted_iota(jnp.int32, (512, 512), 1)
    lt = (qq < rr).astype(jnp.float32)
    offs = jnp.dot(lt, rs, preferred_element_type=jnp.float32)
    rank_excl = (prow + offs - mf).astype(jnp.int32)

    s_total = jnp.sum(mf).astype(jnp.int32)
    u_rank = idx - rank_excl
    pad_pos = s_total + u_rank
    pos_ref[...] = jnp.where(
        mask, rank_excl, jnp.where(pad_pos < TOTAL, pad_pos, TOTAL + (idx >> 12))
    )
    val_ref[...] = jnp.where(mask, idx, N - 1)


_select = pl.pallas_call(
    _select_kernel,
    out_shape=(
        jax.ShapeDtypeStruct((512, 128), jnp.int32),
        jax.ShapeDtypeStruct((512, 128), jnp.int32),
    ),
)


@functools.partial(
    pl.kernel,
    out_type=jax.ShapeDtypeStruct((TOTAL, D), jnp.float32),
    mesh=plsc.VectorSubcoreMesh(core_axis_name="c", subcore_axis_name="s"),
    scratch_types=[
        pltpu.VMEM((32, 128), jnp.int32),
        pltpu.VMEM((32, 128), jnp.int32),
        pltpu.VMEM((RPW,), jnp.int32),
        pltpu.VMEM((CHUNK, D), jnp.float32),
        pltpu.VMEM((CHUNK, D), jnp.float32),
        pltpu.VMEM_SHARED((TOTAL + NS,), jnp.int32),
        pltpu.SemaphoreType.DMA,
        pltpu.SemaphoreType.DMA,
        pltpu.SemaphoreType.DMA,
    ],
)
def _prune_gather(pos_hbm, val_hbm, table_hbm, out_hbm,
                  pos2d, val2d, idxa, rows0, rows1, idx_sh,
                  gsem, wsem0, wsem1):
    cid = lax.axis_index("c")
    sid = lax.axis_index("s")

    pltpu.sync_copy(pos_hbm.at[sid], pos2d)
    pltpu.sync_copy(val_hbm.at[sid], val2d)
    descs = [
        pltpu.async_copy(val2d.at[r], idx_sh.at[pos2d.at[r]], gsem)
        for r in range(32)
    ]
    for d in descs:
        d.wait()
    plsc.subcore_barrier()

    wid = sid * NC + cid
    base = wid * RPW
    pltpu.sync_copy(idx_sh.at[pl.ds(base, RPW)], idxa)
    bufs = ((rows0, wsem0), (rows1, wsem1))
    wb = [None, None]
    for ch in range(NCHUNK):
        b = ch & 1
        rows, wsem = bufs[b]
        if wb[b] is not None:
            wb[b].wait()
        off = base + ch * CHUNK
        idxc = idxa.at[pl.ds(ch * CHUNK, CHUNK)]
        pltpu.async_copy(table_hbm.at[idxc], rows, gsem).wait()
        wb[b] = pltpu.async_copy(rows, out_hbm.at[pl.ds(off, CHUNK)], wsem)
    for d in wb:
        if d is not None:
            d.wait()


def kernel(attention_scores, local_img_fea):
    pos, val = _select(attention_scores.reshape(512, 128))
    pos3 = pos.reshape(NS, 32, 128)
    val3 = val.reshape(NS, 32, 128)
    return _prune_gather(pos3, val3, local_img_fea)
